# baseline (speedup 1.0000x reference)
import jax
import jax.numpy as jnp
from jax.experimental import pallas as pl
from jax.experimental.pallas import tpu as pltpu

_B, _H, _N, _TOPK = 128, 2048, 32768, 64
_Hh = _H // 2
_BN = 2048
_RB = 8


def _hg_kernel(z_ref, w1_ref, b1_ref, gamma_ref, beta_ref, out_ref):
    h = jax.lax.dot_general(z_ref[...], w1_ref[...], (((1,), (1,)), ((), ())),
                            preferred_element_type=jnp.float32)
    h = h + b1_ref[...]
    mu = jnp.mean(h, axis=-1, keepdims=True)
    var = jnp.mean((h - mu) ** 2, axis=-1, keepdims=True)
    hn = (h - mu) / jnp.sqrt(var + 1e-5) * gamma_ref[...] + beta_ref[...]
    out_ref[...] = 0.5 * hn * (1.0 + jax.lax.erf(hn * (1.0 / jnp.sqrt(jnp.float32(2.0)))))


def _scores_kernel(hg_ref, w2_ref, b2_ref, comp_ref, ema_ref, out_ref):
    s = jax.lax.dot_general(hg_ref[...], w2_ref[...], (((1,), (1,)), ((), ())),
                            preferred_element_type=jnp.float32)
    bias = b2_ref[...] + comp_ref[...] * 0.3 + (1.0 / (ema_ref[...] + 1e-6)) * 0.1
    out_ref[...] = s + bias


def _key(x):
    b = jax.lax.bitcast_convert_type(x, jnp.int32)
    return jnp.where(b < 0, b ^ jnp.int32(0x7FFFFFFF), b)


def _ceil_avg(lo, hi):
    return (lo >> 1) + (hi >> 1) + (lo & hi & 1) + ((lo ^ hi) & 1)


def _select_kernel(s_ref, mask_ref, keys):
    s = s_ref[...]
    keys[...] = _key(s)
    kf = jnp.float32(_TOPK)

    cm = s[:, 0:128]
    for j in range(1, _N // 128):
        cm = jnp.maximum(cm, s[:, 128 * j:128 * (j + 1)])
    cmk = _key(cm)
    chi0 = jnp.max(cmk, axis=1, keepdims=True)
    clo = jnp.min(cmk, axis=1, keepdims=True)
    chi = chi0

    def cbody(t, carry):
        lo, hi = carry
        mid = _ceil_avg(lo, hi)
        cnt = jnp.sum(jnp.where(cmk >= mid, 1.0, 0.0), axis=1, keepdims=True)
        ok = cnt >= kf
        return jnp.where(ok, mid, lo), jnp.where(ok, hi, mid - 1)

    clo, chi = jax.lax.fori_loop(0, 32, cbody, (clo, chi))

    def wcond(carry):
        lo, hi = carry
        return jnp.any(lo < hi)

    def wbody(carry):
        lo, hi = carry
        mid = _ceil_avg(lo, hi)
        cnt = jnp.sum(jnp.where(keys[...] >= mid, 1.0, 0.0), axis=1, keepdims=True)
        ok = cnt >= kf
        return jnp.where(ok, mid, lo), jnp.where(ok, hi, mid - 1)

    lo, hi = jax.lax.while_loop(wcond, wbody, (clo, chi0))
    thr = lo

    kk = keys[...]
    cnt_ge = jnp.sum(jnp.where(kk >= thr, 1.0, 0.0), axis=1, keepdims=True)
    anyties = jnp.any(cnt_ge > kf)

    @pl.when(jnp.logical_not(anyties))
    def _():
        mask_ref[...] = jnp.where(kk >= thr, 1.0, 0.0)

    @pl.when(anyties)
    def _():
        iota = jax.lax.broadcasted_iota(jnp.int32, (_RB, _N), 1)
        cnt_gt = jnp.sum(jnp.where(kk > thr, 1.0, 0.0), axis=1, keepdims=True)
        need_eq = kf - cnt_gt
        eq = kk == thr

        ilo = jnp.zeros((_RB, 1), jnp.int32)
        ihi = jnp.full((_RB, 1), _N - 1, jnp.int32)

        def ibody(t, carry):
            ilo, ihi = carry
            mid = (ilo + ihi) >> 1
            cnt = jnp.sum(jnp.where(eq & (iota <= mid), 1.0, 0.0), axis=1,
                          keepdims=True)
            ok = cnt >= need_eq
            return jnp.where(ok, ilo, mid + 1), jnp.where(ok, mid, ihi)

        ilo, ihi = jax.lax.fori_loop(0, 15, ibody, (ilo, ihi))
        mask_ref[...] = jnp.where((kk > thr) | (eq & (iota <= ilo)), 1.0, 0.0)


def _order_kernel(s_ref, idx_ref, cur):
    cur[...] = s_ref[...]
    r_iota = jax.lax.broadcasted_iota(jnp.int32, (_N // 128, 128), 0)
    c_iota = jax.lax.broadcasted_iota(jnp.int32, (_N // 128, 128), 1)
    gidx = r_iota * 128 + c_iota
    kiota = jax.lax.broadcasted_iota(jnp.int32, (8, _TOPK), 1)
    neg_inf = jnp.float32(-jnp.inf)
    idx_ref[...] = jnp.zeros((8, _TOPK), jnp.int32)

    def body(t, carry):
        c = cur[...]
        m = jnp.max(c)
        sel = jnp.min(jnp.where(c == m, gidx, _N))
        idx_ref[...] = jnp.where(kiota == t, sel, idx_ref[...])
        cur[...] = jnp.where(gidx == sel, neg_inf, c)
        return carry

    jax.lax.fori_loop(0, _TOPK, body, 0)


def kernel(z, W1, b1, gamma, beta, W2, b2, competence, activation_ema):
    b1r = b1.reshape(1, _Hh)
    gammar = gamma.reshape(1, _Hh)
    betar = beta.reshape(1, _Hh)
    b2r = b2.reshape(1, _N)
    compr = competence.reshape(1, _N)
    emar = activation_ema.reshape(1, _N)

    hg = pl.pallas_call(
        _hg_kernel,
        out_shape=jax.ShapeDtypeStruct((_B, _Hh), jnp.float32),
    )(z, W1, b1r, gammar, betar)

    grid_n = _N // _BN
    final_scores = pl.pallas_call(
        _scores_kernel,
        grid=(grid_n,),
        in_specs=[
            pl.BlockSpec((_B, _Hh), lambda i: (0, 0)),
            pl.BlockSpec((_BN, _Hh), lambda i: (i, 0)),
            pl.BlockSpec((1, _BN), lambda i: (0, i)),
            pl.BlockSpec((1, _BN), lambda i: (0, i)),
            pl.BlockSpec((1, _BN), lambda i: (0, i)),
        ],
        out_specs=pl.BlockSpec((_B, _BN), lambda i: (0, i)),
        out_shape=jax.ShapeDtypeStruct((_B, _N), jnp.float32),
    )(hg, W2, b2r, compr, emar)

    grid_b = _B // _RB
    mask = pl.pallas_call(
        _select_kernel,
        grid=(grid_b,),
        in_specs=[pl.BlockSpec((_RB, _N), lambda i: (i, 0))],
        out_specs=pl.BlockSpec((_RB, _N), lambda i: (i, 0)),
        out_shape=jax.ShapeDtypeStruct((_B, _N), jnp.float32),
        scratch_shapes=[pltpu.VMEM((_RB, _N), jnp.int32)],
    )(final_scores)

    row0 = final_scores[0].reshape(_N // 128, 128)
    top_idx = pl.pallas_call(
        _order_kernel,
        out_shape=jax.ShapeDtypeStruct((8, _TOPK), jnp.int32),
        scratch_shapes=[pltpu.VMEM((_N // 128, 128), jnp.float32)],
    )(row0)

    selected_indices = top_idx[0]
    return (mask, selected_indices, final_scores)
